# baseline (speedup 1.0000x reference)
import functools
import numpy as np
import jax
import jax.numpy as jnp
from jax.experimental import pallas as pl
from jax.experimental.pallas import tpu as pltpu
from jax.experimental.pallas import tpu_sc as plsc

HID = 128
NFREQ = 10
DIS = NFREQ * 2 * 3
FD_PAD = 64
BE = 4096


_SC_MESH = plsc.VectorSubcoreMesh(core_axis_name="c", subcore_axis_name="s")
GW = 128


def _sc_gather2(hs, hd, src2, dst2):
    E = src2.shape[0] * GW
    n_win = E // GW
    per_tile = n_win // 16
    stage_rows = N2 // 16

    @functools.partial(
        pl.kernel,
        out_type=[jax.ShapeDtypeStruct((E, HID), jnp.float32),
                  jax.ShapeDtypeStruct((E, HID), jnp.float32)],
        mesh=_SC_MESH,
        scratch_types=[
            pltpu.VMEM((2, GW, HID), jnp.float32),
            pltpu.VMEM((2, GW), jnp.int32),
            pltpu.VMEM_SHARED((N2, HID), jnp.float32),
            pltpu.SemaphoreType.DMA,
            pltpu.SemaphoreType.DMA,
        ],
    )
    def k(hs_hbm, hd_hbm, si_hbm, di_hbm, so_hbm, do_hbm,
          buf, idx, table, gsem, osem):
        c = jax.lax.axis_index("c")
        s = jax.lax.axis_index("s")
        r0 = s * stage_rows

        def run(t_hbm, i_hbm, o_hbm):
            pltpu.sync_copy(t_hbm.at[pl.ds(r0, stage_rows)],
                            table.at[pl.ds(r0, stage_rows)])
            plsc.subcore_barrier()
            base = s * per_tile

            @pl.loop(0, per_tile, step=2)
            def _(j):
                w0 = base + j
                pltpu.sync_copy(i_hbm.at[pl.ds(w0, 2)], idx)

                @pl.when(j > 0)
                def _():
                    for b in (0, 1):
                        pltpu.make_async_copy(
                            buf.at[b], o_hbm.at[pl.ds((w0 + b) * GW, GW)], osem
                        ).wait()

                for b in (0, 1):
                    pltpu.async_copy(table.at[idx.at[b]], buf.at[b], gsem)
                for b in (0, 1):
                    pltpu.make_async_copy(table.at[idx.at[b]], buf.at[b], gsem).wait()
                for b in (0, 1):
                    pltpu.async_copy(buf.at[b], o_hbm.at[pl.ds((w0 + b) * GW, GW)], osem)

            for b in (0, 1):
                pltpu.make_async_copy(buf.at[b], o_hbm.at[pl.ds(base * GW, GW)], osem).wait()

        @pl.when(c == 0)
        def _():
            run(hs_hbm, si_hbm, so_hbm)

        @pl.when(c == 1)
        def _():
            run(hd_hbm, di_hbm, do_hbm)

    return k(hs, hd, src2, dst2)


CW = 128
N2 = 10112


def _sc_scatter_add(ef, si2, zeros):
    E2 = ef.shape[0]
    n_chunks = E2 // CW
    per_tile = n_chunks // 32
    rows_per_sub = N2 // 16

    @functools.partial(
        pl.kernel,
        out_type=jax.ShapeDtypeStruct((2, N2, HID), jnp.float32),
        mesh=_SC_MESH,
        scratch_types=[
            pltpu.VMEM((2, CW, HID), jnp.float32),
            pltpu.VMEM((2, CW), jnp.int32),
            pltpu.VMEM_SHARED((N2, HID), jnp.float32),
            pltpu.SemaphoreType.DMA,
            pltpu.SemaphoreType.DMA,
        ],
    )
    def k(ef_hbm, si_hbm, z_hbm, out_hbm, efv, iv, shared, lsem, asem):
        c = jax.lax.axis_index("c")
        s = jax.lax.axis_index("s")
        wid = s * 2 + c
        row0 = s * rows_per_sub
        pltpu.sync_copy(z_hbm.at[pl.ds(row0, rows_per_sub)],
                        shared.at[pl.ds(row0, rows_per_sub)])
        plsc.subcore_barrier()
        base = wid * per_tile

        @pl.loop(0, per_tile, step=2)
        def _(j):
            ch = base + j
            @pl.when(j > 0)
            def _():
                for b in (0, 1):
                    pltpu.make_async_copy(efv.at[b], shared.at[iv.at[b]], asem).wait()
            pltpu.sync_copy(si_hbm.at[pl.ds(ch, 2)], iv)
            for b in (0, 1):
                pltpu.async_copy(ef_hbm.at[pl.ds((ch + b) * CW, CW)], efv.at[b], lsem)
            for b in (0, 1):
                pltpu.make_async_copy(ef_hbm.at[pl.ds((ch + b) * CW, CW)], efv.at[b], lsem).wait()
            for b in (0, 1):
                pltpu.async_copy(efv.at[b], shared.at[iv.at[b]], asem, add=True)

        for b in (0, 1):
            pltpu.make_async_copy(efv.at[b], shared.at[iv.at[b]], asem).wait()
        plsc.subcore_barrier()
        pltpu.sync_copy(shared.at[pl.ds(row0, rows_per_sub)],
                        out_hbm.at[c].at[pl.ds(row0, rows_per_sub)])

    return k(ef, si2, zeros)


def _pack_bf16(x):
    K = x.shape[1] // 2
    xb = x.astype(jnp.bfloat16)
    return jax.lax.bitcast_convert_type(
        jnp.stack([xb[:, :K], xb[:, K:]], axis=-1), jnp.float32)


def _unpack_bf16(p):
    wi = jax.lax.bitcast_convert_type(p, jnp.int32)
    lo = jax.lax.bitcast_convert_type(wi << 16, jnp.float32)
    hi = jax.lax.bitcast_convert_type(jnp.bitwise_and(wi, jnp.int32(-65536)),
                                      jnp.float32)
    return jnp.concatenate([lo, hi], axis=-1)


def _edge_mlp_body(s_ref, d_ref, fd_ref, w1f_ref, w2_ref, b2_ref, out_ref):
    fd = _unpack_bf16(fd_ref[...])
    fdw = jnp.dot(fd.astype(jnp.bfloat16), w1f_ref[...].astype(jnp.bfloat16),
                  preferred_element_type=jnp.float32)
    pre = (s_ref[...] + d_ref[...] + fdw).astype(jnp.bfloat16)
    u = pre * jax.nn.sigmoid(pre)
    v = jnp.dot(u, w2_ref[...].astype(jnp.bfloat16),
                preferred_element_type=jnp.float32) + b2_ref[...]
    vb = v.astype(jnp.bfloat16)
    out_ref[...] = (vb * jax.nn.sigmoid(vb)).astype(jnp.float32)


def _edge_mlp(s_rows, d_rows, fd, w1f, w2, b2):
    E = s_rows.shape[0]
    grid = (E // BE,)
    return pl.pallas_call(
        _edge_mlp_body,
        grid=grid,
        in_specs=[
            pl.BlockSpec((BE, HID), lambda i: (i, 0)),
            pl.BlockSpec((BE, HID), lambda i: (i, 0)),
            pl.BlockSpec((BE, FD_PAD // 2), lambda i: (i, 0)),
            pl.BlockSpec((FD_PAD, HID), lambda i: (0, 0)),
            pl.BlockSpec((HID, HID), lambda i: (0, 0)),
            pl.BlockSpec((1, HID), lambda i: (0, 0)),
        ],
        out_specs=pl.BlockSpec((BE, HID), lambda i: (i, 0)),
        out_shape=jax.ShapeDtypeStruct((E, HID), jnp.float32),
    )(s_rows, d_rows, fd, w1f, w2, b2)


def _node_mlp_body(h_ref, p0_ref, p1_ref, ic_ref, w1a_ref, w1b_ref, b1_ref,
                   w2_ref, b2_ref, out_ref):
    agg = (p0_ref[...] + p1_ref[...]) * ic_ref[...]
    pre = (jnp.dot(h_ref[...].astype(jnp.bfloat16), w1a_ref[...].astype(jnp.bfloat16),
                   preferred_element_type=jnp.float32)
           + jnp.dot(agg.astype(jnp.bfloat16), w1b_ref[...].astype(jnp.bfloat16),
                     preferred_element_type=jnp.float32)
           + b1_ref[...])
    u = pre * jax.nn.sigmoid(pre)
    v = jnp.dot(u.astype(jnp.bfloat16), w2_ref[...].astype(jnp.bfloat16),
                preferred_element_type=jnp.float32) + b2_ref[...]
    out_ref[...] = h_ref[...] + v * jax.nn.sigmoid(v)


def _node_mlp(h, p0, p1, inv_counts, w1a, w1b, b1, w2, b2):
    N = h.shape[0]
    BN = 2000
    grid = (N // BN,)
    return pl.pallas_call(
        _node_mlp_body,
        grid=grid,
        in_specs=[
            pl.BlockSpec((BN, HID), lambda i: (i, 0)),
            pl.BlockSpec((BN, HID), lambda i: (i, 0)),
            pl.BlockSpec((BN, HID), lambda i: (i, 0)),
            pl.BlockSpec((BN, 1), lambda i: (i, 0)),
            pl.BlockSpec((HID, HID), lambda i: (0, 0)),
            pl.BlockSpec((HID, HID), lambda i: (0, 0)),
            pl.BlockSpec((1, HID), lambda i: (0, 0)),
            pl.BlockSpec((HID, HID), lambda i: (0, 0)),
            pl.BlockSpec((1, HID), lambda i: (0, 0)),
        ],
        out_specs=pl.BlockSpec((BN, HID), lambda i: (i, 0)),
        out_shape=jax.ShapeDtypeStruct((N, HID), jnp.float32),
    )(h, p0, p1, inv_counts, w1a, w1b, b1, w2, b2)


def kernel(t, bb_embs, frac_coords, so3_vecs, lattices, node2graph, edge_index,
           W_emb, b_emb, W_lat, b_lat, eW1, eb1, eW2, eb2, nW1, nb1, nW2, nb2,
           W_coord):
    NL = eW1.shape[0]
    N = bb_embs.shape[0]
    E = edge_index.shape[1]
    src = edge_index[0].astype(jnp.int32)
    dst = edge_index[1].astype(jnp.int32)
    n2g = node2graph.astype(jnp.int32)
    E2 = 327680
    pad = E2 - E
    src_g = jnp.pad(src, (0, pad))
    dst_g = jnp.pad(dst, (0, pad))
    src2 = src_g.reshape(E2 // GW, GW)
    dst2 = dst_g.reshape(E2 // GW, GW)
    src_s = jnp.pad(src, (0, pad), constant_values=N2 - 1)
    si2 = src_s.reshape(E2 // CW, CW)
    zeros_n2 = jnp.zeros((N2, HID), jnp.float32)

    freqs = 2.0 * np.pi * jnp.arange(NFREQ, dtype=jnp.float32)
    frac_diff = (frac_coords[dst_g] - frac_coords[src_g]) % 1.0
    emb = (frac_diff[..., None] * freqs).reshape(E2, NFREQ * 3)
    fd = jnp.concatenate([jnp.sin(emb), jnp.cos(emb)], axis=-1)
    fd = jnp.pad(fd, ((0, 0), (0, FD_PAD - DIS)))
    fd = _pack_bf16(fd)

    hemb = bb_embs @ W_emb + b_emb
    so3f = so3_vecs.reshape(N, 16)
    t_per_atom = t[n2g]
    h = jnp.concatenate([hemb, so3f, t_per_atom], axis=1) @ W_lat + b_lat

    counts = jnp.maximum(
        jax.ops.segment_sum(jnp.ones((E,), jnp.float32), src, num_segments=N), 1.0)
    inv_counts = (1.0 / counts)[:, None]

    latW = lattices
    for i in range(NL):
        W1s = eW1[i, :HID]
        W1d = eW1[i, HID:2 * HID]
        W1lat = eW1[i, 2 * HID:2 * HID + 6]
        W1f = jnp.pad(eW1[i, 2 * HID + 6:], ((0, FD_PAD - DIS), (0, 0)))
        lat_term = (latW @ W1lat)[n2g]
        hs = h @ W1s + eb1[i] + lat_term
        hd = h @ W1d
        hs_p = jnp.pad(hs, ((0, N2 - N), (0, 0)))
        hd_p = jnp.pad(hd, ((0, N2 - N), (0, 0)))
        s_rows, d_rows = _sc_gather2(hs_p, hd_p, src2, dst2)
        ef = _edge_mlp(s_rows, d_rows, fd, W1f, eW2[i], eb2[i][None])
        parts = _sc_scatter_add(ef, si2, zeros_n2)
        h = _node_mlp(h, parts[0, :N], parts[1, :N], inv_counts,
                      nW1[i, :HID], nW1[i, HID:], nb1[i][None],
                      nW2[i], nb2[i][None])
    return h @ W_coord


# speedup vs baseline: 1.0234x; 1.0234x over previous
import functools
import numpy as np
import jax
import jax.numpy as jnp
from jax.experimental import pallas as pl
from jax.experimental.pallas import tpu as pltpu
from jax.experimental.pallas import tpu_sc as plsc

HID = 128
NFREQ = 10
DIS = NFREQ * 2 * 3
FD_PAD = 64
BE = 4096


_SC_MESH = plsc.VectorSubcoreMesh(core_axis_name="c", subcore_axis_name="s")
GW = 128


def _sc_gather2(hs, hd, src2, dst2):
    E = src2.shape[0] * GW
    n_win = E // GW
    per_tile = n_win // 16
    stage_rows = N2 // 16

    @functools.partial(
        pl.kernel,
        out_type=[jax.ShapeDtypeStruct((E, HID), jnp.float32),
                  jax.ShapeDtypeStruct((E, HID), jnp.float32)],
        mesh=_SC_MESH,
        scratch_types=[
            pltpu.VMEM((2, GW, HID), jnp.float32),
            pltpu.VMEM((2, GW), jnp.int32),
            pltpu.VMEM_SHARED((N2, HID), jnp.float32),
            pltpu.SemaphoreType.DMA,
            pltpu.SemaphoreType.DMA,
        ],
    )
    def k(hs_hbm, hd_hbm, si_hbm, di_hbm, so_hbm, do_hbm,
          buf, idx, table, gsem, osem):
        c = jax.lax.axis_index("c")
        s = jax.lax.axis_index("s")
        r0 = s * stage_rows

        def run(t_hbm, i_hbm, o_hbm):
            pltpu.sync_copy(t_hbm.at[pl.ds(r0, stage_rows)],
                            table.at[pl.ds(r0, stage_rows)])
            plsc.subcore_barrier()
            base = s * per_tile

            @pl.loop(0, per_tile, step=2)
            def _(j):
                w0 = base + j
                pltpu.sync_copy(i_hbm.at[pl.ds(w0, 2)], idx)

                @pl.when(j > 0)
                def _():
                    for b in (0, 1):
                        pltpu.make_async_copy(
                            buf.at[b], o_hbm.at[pl.ds((w0 + b) * GW, GW)], osem
                        ).wait()

                for b in (0, 1):
                    pltpu.async_copy(table.at[idx.at[b]], buf.at[b], gsem)
                for b in (0, 1):
                    pltpu.make_async_copy(table.at[idx.at[b]], buf.at[b], gsem).wait()
                for b in (0, 1):
                    pltpu.async_copy(buf.at[b], o_hbm.at[pl.ds((w0 + b) * GW, GW)], osem)

            for b in (0, 1):
                pltpu.make_async_copy(buf.at[b], o_hbm.at[pl.ds(base * GW, GW)], osem).wait()

        @pl.when(c == 0)
        def _():
            run(hs_hbm, si_hbm, so_hbm)

        @pl.when(c == 1)
        def _():
            run(hd_hbm, di_hbm, do_hbm)

    return k(hs, hd, src2, dst2)


CW = 128
N2 = 10112


def _sc_scatter_add(ef, si2, zeros):
    E2 = ef.shape[0]
    n_chunks = E2 // CW
    per_tile = n_chunks // 32
    rows_per_sub = N2 // 16

    @functools.partial(
        pl.kernel,
        out_type=jax.ShapeDtypeStruct((2, N2, HID), jnp.float32),
        mesh=_SC_MESH,
        scratch_types=[
            pltpu.VMEM((2, CW, HID), jnp.float32),
            pltpu.VMEM((2, CW), jnp.int32),
            pltpu.VMEM_SHARED((N2, HID), jnp.float32),
            pltpu.SemaphoreType.DMA,
            pltpu.SemaphoreType.DMA,
        ],
    )
    def k(ef_hbm, si_hbm, z_hbm, out_hbm, efv, iv, shared, lsem, asem):
        c = jax.lax.axis_index("c")
        s = jax.lax.axis_index("s")
        wid = s * 2 + c
        row0 = s * rows_per_sub
        pltpu.sync_copy(z_hbm.at[pl.ds(row0, rows_per_sub)],
                        shared.at[pl.ds(row0, rows_per_sub)])
        plsc.subcore_barrier()
        base = wid * per_tile

        @pl.loop(0, per_tile, step=2)
        def _(j):
            ch = base + j
            @pl.when(j > 0)
            def _():
                for b in (0, 1):
                    pltpu.make_async_copy(efv.at[b], shared.at[iv.at[b]], asem).wait()
            pltpu.sync_copy(si_hbm.at[pl.ds(ch, 2)], iv)
            for b in (0, 1):
                pltpu.async_copy(ef_hbm.at[pl.ds((ch + b) * CW, CW)], efv.at[b], lsem)
            for b in (0, 1):
                pltpu.make_async_copy(ef_hbm.at[pl.ds((ch + b) * CW, CW)], efv.at[b], lsem).wait()
            for b in (0, 1):
                pltpu.async_copy(efv.at[b], shared.at[iv.at[b]], asem, add=True)

        for b in (0, 1):
            pltpu.make_async_copy(efv.at[b], shared.at[iv.at[b]], asem).wait()
        plsc.subcore_barrier()
        pltpu.sync_copy(shared.at[pl.ds(row0, rows_per_sub)],
                        out_hbm.at[c].at[pl.ds(row0, rows_per_sub)])

    return k(ef, si2, zeros)


def _pack_bf16(x):
    K = x.shape[1] // 2
    xb = x.astype(jnp.bfloat16)
    return jax.lax.bitcast_convert_type(
        jnp.stack([xb[:, :K], xb[:, K:]], axis=-1), jnp.float32)


def _unpack_bf16(p):
    wi = jax.lax.bitcast_convert_type(p, jnp.int32)
    lo = jax.lax.bitcast_convert_type(wi << 16, jnp.float32)
    hi = jax.lax.bitcast_convert_type(jnp.bitwise_and(wi, jnp.int32(-65536)),
                                      jnp.float32)
    return jnp.concatenate([lo, hi], axis=-1)


def _edge_mlp_body(s_ref, d_ref, fd_ref, w1f_ref, w2_ref, b2_ref, out_ref):
    fd = _unpack_bf16(fd_ref[...])
    fdw = jnp.dot(fd.astype(jnp.bfloat16), w1f_ref[...].astype(jnp.bfloat16),
                  preferred_element_type=jnp.float32)
    pre = (s_ref[...] + d_ref[...] + fdw).astype(jnp.bfloat16)
    u = pre * jax.nn.sigmoid(pre)
    v = jnp.dot(u, w2_ref[...].astype(jnp.bfloat16),
                preferred_element_type=jnp.float32) + b2_ref[...]
    vb = v.astype(jnp.bfloat16)
    out_ref[...] = (vb * jax.nn.sigmoid(vb)).astype(jnp.float32)


def _edge_mlp(s_rows, d_rows, fd, w1f, w2, b2):
    E = s_rows.shape[0]
    grid = (E // BE,)
    return pl.pallas_call(
        _edge_mlp_body,
        grid=grid,
        in_specs=[
            pl.BlockSpec((BE, HID), lambda i: (i, 0)),
            pl.BlockSpec((BE, HID), lambda i: (i, 0)),
            pl.BlockSpec((BE, FD_PAD // 2), lambda i: (i, 0)),
            pl.BlockSpec((FD_PAD, HID), lambda i: (0, 0)),
            pl.BlockSpec((HID, HID), lambda i: (0, 0)),
            pl.BlockSpec((1, HID), lambda i: (0, 0)),
        ],
        out_specs=pl.BlockSpec((BE, HID), lambda i: (i, 0)),
        out_shape=jax.ShapeDtypeStruct((E, HID), jnp.float32),
    )(s_rows, d_rows, fd, w1f, w2, b2)


def _node_mlp_body(h_ref, p0_ref, p1_ref, p2_ref, p3_ref, ic_ref, w1a_ref,
                   w1b_ref, b1_ref, w2_ref, b2_ref, out_ref):
    agg = (p0_ref[...] + p1_ref[...] + p2_ref[...] + p3_ref[...]) * ic_ref[...]
    pre = (jnp.dot(h_ref[...].astype(jnp.bfloat16), w1a_ref[...].astype(jnp.bfloat16),
                   preferred_element_type=jnp.float32)
           + jnp.dot(agg.astype(jnp.bfloat16), w1b_ref[...].astype(jnp.bfloat16),
                     preferred_element_type=jnp.float32)
           + b1_ref[...])
    u = pre * jax.nn.sigmoid(pre)
    v = jnp.dot(u.astype(jnp.bfloat16), w2_ref[...].astype(jnp.bfloat16),
                preferred_element_type=jnp.float32) + b2_ref[...]
    out_ref[...] = h_ref[...] + v * jax.nn.sigmoid(v)


def _node_mlp(h, p0, p1, p2, p3, inv_counts, w1a, w1b, b1, w2, b2):
    N = h.shape[0]
    BN = 2000
    grid = (N // BN,)
    return pl.pallas_call(
        _node_mlp_body,
        grid=grid,
        in_specs=[
            pl.BlockSpec((BN, HID), lambda i: (i, 0)),
            pl.BlockSpec((BN, HID), lambda i: (i, 0)),
            pl.BlockSpec((BN, HID), lambda i: (i, 0)),
            pl.BlockSpec((BN, HID), lambda i: (i, 0)),
            pl.BlockSpec((BN, HID), lambda i: (i, 0)),
            pl.BlockSpec((BN, 1), lambda i: (i, 0)),
            pl.BlockSpec((HID, HID), lambda i: (0, 0)),
            pl.BlockSpec((HID, HID), lambda i: (0, 0)),
            pl.BlockSpec((1, HID), lambda i: (0, 0)),
            pl.BlockSpec((HID, HID), lambda i: (0, 0)),
            pl.BlockSpec((1, HID), lambda i: (0, 0)),
        ],
        out_specs=pl.BlockSpec((BN, HID), lambda i: (i, 0)),
        out_shape=jax.ShapeDtypeStruct((N, HID), jnp.float32),
    )(h, p0, p1, p2, p3, inv_counts, w1a, w1b, b1, w2, b2)


def kernel(t, bb_embs, frac_coords, so3_vecs, lattices, node2graph, edge_index,
           W_emb, b_emb, W_lat, b_lat, eW1, eb1, eW2, eb2, nW1, nb1, nW2, nb2,
           W_coord):
    NL = eW1.shape[0]
    N = bb_embs.shape[0]
    E = edge_index.shape[1]
    src = edge_index[0].astype(jnp.int32)
    dst = edge_index[1].astype(jnp.int32)
    n2g = node2graph.astype(jnp.int32)
    E2 = 327680
    pad = E2 - E
    src_g = jnp.pad(src, (0, pad))
    dst_g = jnp.pad(dst, (0, pad))
    src2 = src_g.reshape(E2 // GW, GW)
    dst2 = dst_g.reshape(E2 // GW, GW)
    src_s = jnp.pad(src, (0, pad), constant_values=N2 - 1)
    si2 = src_s.reshape(E2 // CW, CW)
    zeros_n2 = jnp.zeros((N2, HID), jnp.float32)
    EH = E2 // 2
    hw = EH // GW
    hc = EH // CW

    freqs = 2.0 * np.pi * jnp.arange(NFREQ, dtype=jnp.float32)
    frac_diff = (frac_coords[dst_g] - frac_coords[src_g]) % 1.0
    emb = (frac_diff[..., None] * freqs).reshape(E2, NFREQ * 3)
    fd = jnp.concatenate([jnp.sin(emb), jnp.cos(emb)], axis=-1)
    fd = jnp.pad(fd, ((0, 0), (0, FD_PAD - DIS)))
    fd = _pack_bf16(fd)

    hemb = bb_embs @ W_emb + b_emb
    so3f = so3_vecs.reshape(N, 16)
    t_per_atom = t[n2g]
    h = jnp.concatenate([hemb, so3f, t_per_atom], axis=1) @ W_lat + b_lat

    counts = jnp.maximum(
        jax.ops.segment_sum(jnp.ones((E,), jnp.float32), src, num_segments=N), 1.0)
    inv_counts = (1.0 / counts)[:, None]

    latW = lattices
    for i in range(NL):
        W1s = eW1[i, :HID]
        W1d = eW1[i, HID:2 * HID]
        W1lat = eW1[i, 2 * HID:2 * HID + 6]
        W1f = jnp.pad(eW1[i, 2 * HID + 6:], ((0, FD_PAD - DIS), (0, 0)))
        lat_term = (latW @ W1lat)[n2g]
        hs = h @ W1s + eb1[i] + lat_term
        hd = h @ W1d
        hs_p = jnp.pad(hs, ((0, N2 - N), (0, 0)))
        hd_p = jnp.pad(hd, ((0, N2 - N), (0, 0)))
        pp = []
        for half in (0, 1):
            s_rows, d_rows = _sc_gather2(hs_p, hd_p,
                                         src2[half * hw:(half + 1) * hw],
                                         dst2[half * hw:(half + 1) * hw])
            ef = _edge_mlp(s_rows, d_rows, fd[half * EH:(half + 1) * EH],
                           W1f, eW2[i], eb2[i][None])
            pp.append(_sc_scatter_add(ef, si2[half * hc:(half + 1) * hc],
                                      zeros_n2))
        h = _node_mlp(h, pp[0][0, :N], pp[0][1, :N], pp[1][0, :N], pp[1][1, :N],
                      inv_counts, nW1[i, :HID], nW1[i, HID:], nb1[i][None],
                      nW2[i], nb2[i][None])
    return h @ W_coord


# speedup vs baseline: 1.0276x; 1.0041x over previous
import functools
import numpy as np
import jax
import jax.numpy as jnp
from jax.experimental import pallas as pl
from jax.experimental.pallas import tpu as pltpu
from jax.experimental.pallas import tpu_sc as plsc

HID = 128
NFREQ = 10
DIS = NFREQ * 2 * 3
FD_PAD = 64
BE = 4096


_SC_MESH = plsc.VectorSubcoreMesh(core_axis_name="c", subcore_axis_name="s")
GW = 128


def _sc_gather2(hs, hd, src2, dst2):
    E = src2.shape[0] * GW
    n_win = E // GW
    per_tile = n_win // 16
    stage_rows = N2 // 16

    @functools.partial(
        pl.kernel,
        out_type=[jax.ShapeDtypeStruct((E, HID), jnp.float32),
                  jax.ShapeDtypeStruct((E, HID), jnp.float32)],
        mesh=_SC_MESH,
        scratch_types=[
            pltpu.VMEM((2, GW, HID), jnp.float32),
            pltpu.VMEM((2, GW), jnp.int32),
            pltpu.VMEM_SHARED((N2, HID), jnp.float32),
            pltpu.SemaphoreType.DMA,
            pltpu.SemaphoreType.DMA,
        ],
    )
    def k(hs_hbm, hd_hbm, si_hbm, di_hbm, so_hbm, do_hbm,
          buf, idx, table, gsem, osem):
        c = jax.lax.axis_index("c")
        s = jax.lax.axis_index("s")
        r0 = s * stage_rows

        def run(t_hbm, i_hbm, o_hbm):
            pltpu.sync_copy(t_hbm.at[pl.ds(r0, stage_rows)],
                            table.at[pl.ds(r0, stage_rows)])
            plsc.subcore_barrier()
            base = s * per_tile

            @pl.loop(0, per_tile, step=2)
            def _(j):
                w0 = base + j
                pltpu.sync_copy(i_hbm.at[pl.ds(w0, 2)], idx)

                @pl.when(j > 0)
                def _():
                    for b in (0, 1):
                        pltpu.make_async_copy(
                            buf.at[b], o_hbm.at[pl.ds((w0 + b) * GW, GW)], osem
                        ).wait()

                for b in (0, 1):
                    pltpu.async_copy(table.at[idx.at[b]], buf.at[b], gsem)
                for b in (0, 1):
                    pltpu.make_async_copy(table.at[idx.at[b]], buf.at[b], gsem).wait()
                for b in (0, 1):
                    pltpu.async_copy(buf.at[b], o_hbm.at[pl.ds((w0 + b) * GW, GW)], osem)

            for b in (0, 1):
                pltpu.make_async_copy(buf.at[b], o_hbm.at[pl.ds(base * GW, GW)], osem).wait()

        @pl.when(c == 0)
        def _():
            run(hs_hbm, si_hbm, so_hbm)

        @pl.when(c == 1)
        def _():
            run(hd_hbm, di_hbm, do_hbm)

    return k(hs, hd, src2, dst2)


CW = 128
N2 = 10112


def _sc_scatter_add(ef, si2, zeros):
    E2 = ef.shape[0]
    n_chunks = E2 // CW
    per_tile = n_chunks // 32
    rows_per_sub = N2 // 16

    @functools.partial(
        pl.kernel,
        out_type=jax.ShapeDtypeStruct((2, N2, HID), jnp.float32),
        mesh=_SC_MESH,
        scratch_types=[
            pltpu.VMEM((2, CW, HID), jnp.float32),
            pltpu.VMEM((2, CW), jnp.int32),
            pltpu.VMEM_SHARED((N2, HID), jnp.float32),
            pltpu.SemaphoreType.DMA,
            pltpu.SemaphoreType.DMA,
        ],
    )
    def k(ef_hbm, si_hbm, z_hbm, out_hbm, efv, iv, shared, lsem, asem):
        c = jax.lax.axis_index("c")
        s = jax.lax.axis_index("s")
        wid = s * 2 + c
        row0 = s * rows_per_sub
        pltpu.sync_copy(z_hbm.at[pl.ds(row0, rows_per_sub)],
                        shared.at[pl.ds(row0, rows_per_sub)])
        plsc.subcore_barrier()
        base = wid * per_tile

        @pl.loop(0, per_tile, step=2)
        def _(j):
            ch = base + j
            @pl.when(j > 0)
            def _():
                for b in (0, 1):
                    pltpu.make_async_copy(efv.at[b], shared.at[iv.at[b]], asem).wait()
            pltpu.sync_copy(si_hbm.at[pl.ds(ch, 2)], iv)
            for b in (0, 1):
                pltpu.async_copy(ef_hbm.at[pl.ds((ch + b) * CW, CW)], efv.at[b], lsem)
            for b in (0, 1):
                pltpu.make_async_copy(ef_hbm.at[pl.ds((ch + b) * CW, CW)], efv.at[b], lsem).wait()
            for b in (0, 1):
                pltpu.async_copy(efv.at[b], shared.at[iv.at[b]], asem, add=True)

        for b in (0, 1):
            pltpu.make_async_copy(efv.at[b], shared.at[iv.at[b]], asem).wait()
        plsc.subcore_barrier()
        pltpu.sync_copy(shared.at[pl.ds(row0, rows_per_sub)],
                        out_hbm.at[c].at[pl.ds(row0, rows_per_sub)])

    return k(ef, si2, zeros)


def _pack_bf16(x):
    K = x.shape[1] // 2
    xb = x.astype(jnp.bfloat16)
    return jax.lax.bitcast_convert_type(
        jnp.stack([xb[:, :K], xb[:, K:]], axis=-1), jnp.float32)


def _unpack_bf16(p):
    wi = jax.lax.bitcast_convert_type(p, jnp.int32)
    lo = jax.lax.bitcast_convert_type(wi << 16, jnp.float32)
    hi = jax.lax.bitcast_convert_type(jnp.bitwise_and(wi, jnp.int32(-65536)),
                                      jnp.float32)
    return jnp.concatenate([lo, hi], axis=-1)


def _edge_mlp_body(s_ref, d_ref, fd_ref, w1f_ref, w2_ref, b2_ref, out_ref):
    fd = _unpack_bf16(fd_ref[...])
    fdw = jnp.dot(fd.astype(jnp.bfloat16), w1f_ref[...].astype(jnp.bfloat16),
                  preferred_element_type=jnp.float32)
    pre = (s_ref[...] + d_ref[...] + fdw).astype(jnp.bfloat16)
    u = pre * jax.nn.sigmoid(pre)
    v = jnp.dot(u, w2_ref[...].astype(jnp.bfloat16),
                preferred_element_type=jnp.float32) + b2_ref[...]
    vb = v.astype(jnp.bfloat16)
    out_ref[...] = (vb * jax.nn.sigmoid(vb)).astype(jnp.float32)


def _edge_mlp(s_rows, d_rows, fd, w1f, w2, b2):
    E = s_rows.shape[0]
    grid = (E // BE,)
    return pl.pallas_call(
        _edge_mlp_body,
        grid=grid,
        in_specs=[
            pl.BlockSpec((BE, HID), lambda i: (i, 0)),
            pl.BlockSpec((BE, HID), lambda i: (i, 0)),
            pl.BlockSpec((BE, FD_PAD // 2), lambda i: (i, 0)),
            pl.BlockSpec((FD_PAD, HID), lambda i: (0, 0)),
            pl.BlockSpec((HID, HID), lambda i: (0, 0)),
            pl.BlockSpec((1, HID), lambda i: (0, 0)),
        ],
        out_specs=pl.BlockSpec((BE, HID), lambda i: (i, 0)),
        out_shape=jax.ShapeDtypeStruct((E, HID), jnp.float32),
    )(s_rows, d_rows, fd, w1f, w2, b2)


def _node_mlp_body(h_ref, p0_ref, p1_ref, p2_ref, p3_ref, ic_ref, w1a_ref,
                   w1b_ref, b1_ref, w2_ref, b2_ref, out_ref):
    agg = (p0_ref[...] + p1_ref[...] + p2_ref[...] + p3_ref[...]) * ic_ref[...]
    pre = (jnp.dot(h_ref[...].astype(jnp.bfloat16), w1a_ref[...].astype(jnp.bfloat16),
                   preferred_element_type=jnp.float32)
           + jnp.dot(agg.astype(jnp.bfloat16), w1b_ref[...].astype(jnp.bfloat16),
                     preferred_element_type=jnp.float32)
           + b1_ref[...])
    u = pre * jax.nn.sigmoid(pre)
    v = jnp.dot(u.astype(jnp.bfloat16), w2_ref[...].astype(jnp.bfloat16),
                preferred_element_type=jnp.float32) + b2_ref[...]
    out_ref[...] = h_ref[...] + v * jax.nn.sigmoid(v)


def _node_mlp(h, p0, p1, p2, p3, inv_counts, w1a, w1b, b1, w2, b2):
    N = h.shape[0]
    BN = 1264
    grid = (N // BN,)
    return pl.pallas_call(
        _node_mlp_body,
        grid=grid,
        in_specs=[
            pl.BlockSpec((BN, HID), lambda i: (i, 0)),
            pl.BlockSpec((BN, HID), lambda i: (i, 0)),
            pl.BlockSpec((BN, HID), lambda i: (i, 0)),
            pl.BlockSpec((BN, HID), lambda i: (i, 0)),
            pl.BlockSpec((BN, HID), lambda i: (i, 0)),
            pl.BlockSpec((BN, 1), lambda i: (i, 0)),
            pl.BlockSpec((HID, HID), lambda i: (0, 0)),
            pl.BlockSpec((HID, HID), lambda i: (0, 0)),
            pl.BlockSpec((1, HID), lambda i: (0, 0)),
            pl.BlockSpec((HID, HID), lambda i: (0, 0)),
            pl.BlockSpec((1, HID), lambda i: (0, 0)),
        ],
        out_specs=pl.BlockSpec((BN, HID), lambda i: (i, 0)),
        out_shape=jax.ShapeDtypeStruct((N, HID), jnp.float32),
    )(h, p0, p1, p2, p3, inv_counts, w1a, w1b, b1, w2, b2)


def kernel(t, bb_embs, frac_coords, so3_vecs, lattices, node2graph, edge_index,
           W_emb, b_emb, W_lat, b_lat, eW1, eb1, eW2, eb2, nW1, nb1, nW2, nb2,
           W_coord):
    NL = eW1.shape[0]
    N = bb_embs.shape[0]
    E = edge_index.shape[1]
    src = edge_index[0].astype(jnp.int32)
    dst = edge_index[1].astype(jnp.int32)
    n2g = node2graph.astype(jnp.int32)
    E2 = 327680
    pad = E2 - E
    src_g = jnp.pad(src, (0, pad))
    dst_g = jnp.pad(dst, (0, pad))
    src2 = src_g.reshape(E2 // GW, GW)
    dst2 = dst_g.reshape(E2 // GW, GW)
    src_s = jnp.pad(src, (0, pad), constant_values=N2 - 1)
    si2 = src_s.reshape(E2 // CW, CW)
    zeros_n2 = jnp.zeros((N2, HID), jnp.float32)
    EH = E2 // 2
    hw = EH // GW
    hc = EH // CW

    freqs = 2.0 * np.pi * jnp.arange(NFREQ, dtype=jnp.float32)
    frac_diff = (frac_coords[dst_g] - frac_coords[src_g]) % 1.0
    emb = (frac_diff[..., None] * freqs).reshape(E2, NFREQ * 3)
    fd = jnp.concatenate([jnp.sin(emb), jnp.cos(emb)], axis=-1)
    fd = jnp.pad(fd, ((0, 0), (0, FD_PAD - DIS)))
    fd = _pack_bf16(fd)

    hemb = bb_embs @ W_emb + b_emb
    so3f = so3_vecs.reshape(N, 16)
    t_per_atom = t[n2g]
    h = jnp.concatenate([hemb, so3f, t_per_atom], axis=1) @ W_lat + b_lat
    h = jnp.pad(h, ((0, N2 - N), (0, 0)))

    counts = jnp.maximum(
        jax.ops.segment_sum(jnp.ones((E,), jnp.float32), src, num_segments=N), 1.0)
    inv_counts = jnp.pad((1.0 / counts)[:, None], ((0, N2 - N), (0, 0)),
                         constant_values=1.0)
    n2g_p = jnp.pad(n2g, (0, N2 - N))

    latW = lattices
    for i in range(NL):
        W1s = eW1[i, :HID]
        W1d = eW1[i, HID:2 * HID]
        W1lat = eW1[i, 2 * HID:2 * HID + 6]
        W1f = jnp.pad(eW1[i, 2 * HID + 6:], ((0, FD_PAD - DIS), (0, 0)))
        lat_term = (latW @ W1lat)[n2g_p]
        hs_p = h @ W1s + eb1[i] + lat_term
        hd_p = h @ W1d
        pp = []
        for half in (0, 1):
            s_rows, d_rows = _sc_gather2(hs_p, hd_p,
                                         src2[half * hw:(half + 1) * hw],
                                         dst2[half * hw:(half + 1) * hw])
            ef = _edge_mlp(s_rows, d_rows, fd[half * EH:(half + 1) * EH],
                           W1f, eW2[i], eb2[i][None])
            pp.append(_sc_scatter_add(ef, si2[half * hc:(half + 1) * hc],
                                      zeros_n2))
        h = _node_mlp(h, pp[0][0], pp[0][1], pp[1][0], pp[1][1],
                      inv_counts, nW1[i, :HID], nW1[i, HID:], nb1[i][None],
                      nW2[i], nb2[i][None])
    return h[:N] @ W_coord
